# baseline (speedup 1.0000x reference)
import functools

import jax
import jax.numpy as jnp
from jax import lax
from jax.experimental import pallas as pl
from jax.experimental.pallas import tpu as pltpu
from jax.experimental.pallas import tpu_sc as plsc

N = 10000
C = 128
EPS = 1e-5
N_PAD = 10240
JUNK = N
B = 1024
NB = N_PAD // B

NW = 32
CH = 64
RPT = N_PAD // 16

_MESH = dict(core_axis_name="c", subcore_axis_name="s")


def _sc_deg_body(nchunks, epw, dst_hbm, ones_hbm, zeros_hbm, out_hbm,
                 didx_v, ones_v, acc_sh):
    cid = lax.axis_index("c")
    sid = lax.axis_index("s")
    wid = sid * 2 + cid
    pltpu.sync_copy(zeros_hbm.at[pl.ds(sid * RPT, RPT)],
                    acc_sh.at[pl.ds(sid * RPT, RPT)])
    pltpu.sync_copy(ones_hbm, ones_v)
    plsc.subcore_barrier()
    base = wid * epw

    def body(j, carry):
        off = base + j * CH
        pltpu.sync_copy(dst_hbm.at[pl.ds(off, CH)], didx_v)
        pltpu.sync_copy(ones_v, acc_sh.at[didx_v], add=True)
        return carry

    lax.fori_loop(0, nchunks, body, 0)
    plsc.subcore_barrier()
    pltpu.sync_copy(acc_sh.at[pl.ds(sid * RPT, RPT)],
                    out_hbm.at[pl.ds(cid * N_PAD + sid * RPT, RPT)])


NBUF = 4


def _sc_gs_body(nchunks, epw, src_hbm, dst_hbm, scaled_hbm, zeros_hbm,
                out_hbm, sidx_v, didx_v, rows_v, s0, s1, s2, s3, acc_sh):
    cid = lax.axis_index("c")
    sid = lax.axis_index("s")
    wid = sid * 2 + cid
    sems = (s0, s1, s2, s3)
    base = wid * epw
    pltpu.sync_copy(zeros_hbm.at[pl.ds(sid * RPT, RPT)],
                    acc_sh.at[pl.ds(sid * RPT, RPT)])
    plsc.subcore_barrier()

    def fire(j, b):
        pltpu.sync_copy(src_hbm.at[pl.ds(base + j * CH, CH)], sidx_v.at[b])
        pltpu.async_copy(scaled_hbm.at[sidx_v.at[b]], rows_v.at[b], sems[b])

    for b in range(NBUF - 1):
        fire(b, b)

    def outer(jo, carry):
        for b in range(NBUF):
            j = jo * NBUF + b
            jn = j + NBUF - 1
            nb = (b + NBUF - 1) % NBUF

            @pl.when(jn < nchunks)
            def _():
                fire(jn, nb)

            pltpu.sync_copy(dst_hbm.at[pl.ds(base + j * CH, CH)], didx_v)
            pltpu.make_async_copy(scaled_hbm.at[sidx_v.at[b]],
                                  rows_v.at[b], sems[b]).wait()
            pltpu.sync_copy(rows_v.at[b], acc_sh.at[didx_v], add=True)
        return carry

    lax.fori_loop(0, nchunks // NBUF, outer, 0)
    plsc.subcore_barrier()
    pltpu.sync_copy(acc_sh.at[pl.ds(sid * RPT, RPT)],
                    out_hbm.at[pl.ds(cid * N_PAD + sid * RPT, RPT)])


def _make_sc_deg(nchunks, epw):
    return pl.kernel(
        functools.partial(_sc_deg_body, nchunks, epw),
        out_type=jax.ShapeDtypeStruct((2 * N_PAD, 16), jnp.float32),
        mesh=plsc.VectorSubcoreMesh(**_MESH),
        scratch_types=[
            pltpu.VMEM((CH,), jnp.int32),
            pltpu.VMEM((CH, 16), jnp.float32),
            pltpu.VMEM_SHARED((N_PAD, 16), jnp.float32),
        ],
        compiler_params=pltpu.CompilerParams(use_tc_tiling_on_sc=False),
    )


def _make_sc_gs(nchunks, epw):
    return pl.kernel(
        functools.partial(_sc_gs_body, nchunks, epw),
        out_type=jax.ShapeDtypeStruct((2 * N_PAD, C), jnp.float32),
        mesh=plsc.VectorSubcoreMesh(**_MESH),
        scratch_types=[
            pltpu.VMEM((NBUF, CH), jnp.int32),
            pltpu.VMEM((CH,), jnp.int32),
            pltpu.VMEM((NBUF, CH, C), jnp.float32),
            pltpu.SemaphoreType.DMA,
            pltpu.SemaphoreType.DMA,
            pltpu.SemaphoreType.DMA,
            pltpu.SemaphoreType.DMA,
            pltpu.VMEM_SHARED((N_PAD, C), jnp.float32),
        ],
    )


def _mm_bias_body(x_ref, w_ref, b_ref, o_ref):
    o_ref[...] = jnp.dot(x_ref[...], w_ref[...],
                         preferred_element_type=jnp.float32) + b_ref[...]


def _mm_scale_body(x_ref, w_ref, s_ref, o_ref):
    o_ref[...] = jnp.dot(x_ref[...], w_ref[...],
                         preferred_element_type=jnp.float32) * s_ref[...]


def _dinv_body(d0_ref, d1_ref, o_ref):
    deg = jnp.maximum(d0_ref[...] + d1_ref[...], 1.0)
    o_ref[...] = jnp.broadcast_to(lax.rsqrt(deg)[:, :1], (B, C))


def _bn_body(has_res, r0_ref, r1_ref, s_ref, b_ref, g_ref, be_ref, res_ref,
             o_ref, sum_ref, ssq_ref):
    p = pl.program_id(0)
    i = pl.program_id(1)
    y = s_ref[...] * (r0_ref[...] + r1_ref[...]) + b_ref[...]

    @pl.when(jnp.logical_and(p == 0, i == 0))
    def _():
        sum_ref[...] = jnp.zeros_like(sum_ref)
        ssq_ref[...] = jnp.zeros_like(ssq_ref)

    @pl.when(p == 0)
    def _():
        rows = lax.broadcasted_iota(jnp.int32, (B, C), 0) + i * B
        ym = jnp.where(rows < N, y, 0.0)
        sum_ref[...] += jnp.sum(ym, axis=0, keepdims=True)
        ssq_ref[...] += jnp.sum(ym * ym, axis=0, keepdims=True)

    @pl.when(p == 1)
    def _():
        m = sum_ref[...] * (1.0 / N)
        v = ssq_ref[...] * (1.0 / N) - m * m
        h = (y - m) * lax.rsqrt(v + EPS) * g_ref[...] + be_ref[...]
        if has_res:
            h = h + res_ref[...]
        o_ref[...] = jnp.maximum(h, 0.0)


def _heads_body(h_ref, wm_ref, bm_ref, wl_ref, bl_ref, mu_ref, lv_ref):
    h = h_ref[...]
    mu_ref[...] = jnp.dot(h, wm_ref[...],
                          preferred_element_type=jnp.float32) + bm_ref[...]
    lv_ref[...] = jnp.dot(h, wl_ref[...],
                          preferred_element_type=jnp.float32) + bl_ref[...]


_row_spec = pl.BlockSpec((B, C), lambda i: (i, 0))
_full_w = pl.BlockSpec((C, C), lambda i: (0, 0))
_full_b = pl.BlockSpec((1, C), lambda i: (0, 0))
_out_f32 = jax.ShapeDtypeStruct((N_PAD, C), jnp.float32)


def _mm_bias(x, w, b):
    return pl.pallas_call(
        _mm_bias_body, grid=(NB,),
        in_specs=[_row_spec, _full_w, _full_b],
        out_specs=_row_spec, out_shape=_out_f32,
    )(x, w, b.reshape(1, C))


def _mm_scale(x, w, s):
    return pl.pallas_call(
        _mm_scale_body, grid=(NB,),
        in_specs=[_row_spec, _full_w, _row_spec],
        out_specs=_row_spec, out_shape=_out_f32,
    )(x, w, s)


def _dinv(deg2):
    spec16 = pl.BlockSpec((B, 16), lambda i: (i, 0))
    return pl.pallas_call(
        _dinv_body, grid=(NB,),
        in_specs=[spec16, spec16],
        out_specs=_row_spec, out_shape=_out_f32,
    )(deg2[:N_PAD], deg2[N_PAD:])


def _bn(raw2, dinv, b, g, be, res):
    has_res = res is not None
    spec2 = pl.BlockSpec((B, C), lambda p, i: (i, 0))
    full2 = pl.BlockSpec((1, C), lambda p, i: (0, 0))
    out2 = pl.BlockSpec((B, C), lambda p, i: (jnp.where(p == 1, i, 0), 0))
    if not has_res:
        res = raw2[:N_PAD]
    return pl.pallas_call(
        functools.partial(_bn_body, has_res), grid=(2, NB),
        in_specs=[spec2, spec2, spec2, full2, full2, full2, spec2],
        out_specs=out2, out_shape=_out_f32,
        scratch_shapes=[pltpu.VMEM((1, C), jnp.float32),
                        pltpu.VMEM((1, C), jnp.float32)],
    )(raw2[:N_PAD], raw2[N_PAD:], dinv, b.reshape(1, C), g.reshape(1, C),
      be.reshape(1, C), res)


def _heads(h, wm, bm, wl, bl):
    return pl.pallas_call(
        _heads_body, grid=(NB,),
        in_specs=[_row_spec, _full_w, _full_b, _full_w, _full_b],
        out_specs=(_row_spec, _row_spec), out_shape=(_out_f32, _out_f32),
    )(h, wm, bm.reshape(1, C), wl, bl.reshape(1, C))


def kernel(x, edge_index, W1, b1, W2, b2, W3, b3, W4, b4,
           g1, be1, g2, be2, g3, be3, g4, be4,
           Wr, br, Wmu, bmu, Wlv, blv):
    e = edge_index.shape[1]
    e2 = e + N
    epw = -(-e2 // (NW * CH * NBUF)) * CH * NBUF
    nchunks = epw // CH
    pad = NW * epw - e2

    loop = jnp.arange(N, dtype=jnp.int32)
    src = jnp.concatenate([edge_index[0].astype(jnp.int32), loop,
                           jnp.zeros((pad,), jnp.int32)])
    dst = jnp.concatenate([edge_index[1].astype(jnp.int32), loop,
                           jnp.full((pad,), JUNK, jnp.int32)])

    zeros_big = jnp.zeros((N_PAD, C), jnp.float32)
    x_pad = jnp.zeros((N_PAD, C), x.dtype).at[:N].set(x)

    deg2 = _make_sc_deg(nchunks, epw)(
        dst, jnp.ones((CH, 16), jnp.float32), jnp.zeros((N_PAD, 16),
                                                        jnp.float32))
    dinv = _dinv(deg2)

    identity = _mm_bias(x_pad, Wr, br)

    sc_gs = _make_sc_gs(nchunks, epw)
    h = x_pad
    for W, b, g, be, res in ((W1, b1, g1, be1, None),
                             (W2, b2, g2, be2, None),
                             (W3, b3, g3, be3, None),
                             (W4, b4, g4, be4, identity)):
        scaled = _mm_scale(h, W, dinv)
        raw2 = sc_gs(src, dst, scaled, zeros_big)
        h = _bn(raw2, dinv, b, g, be, res)

    mu, lv = _heads(h, Wmu, bmu, Wlv, blv)
    return mu[:N], lv[:N]


# speedup vs baseline: 1.8573x; 1.8573x over previous
import functools

import jax
import jax.numpy as jnp
from jax import lax
from jax.experimental import pallas as pl
from jax.experimental.pallas import tpu as pltpu
from jax.experimental.pallas import tpu_sc as plsc

N = 10000
C = 128
EPS = 1e-5
N_PAD = 10240
JUNK = N
B = 1024
NB = N_PAD // B

NW = 32
CH = 128
RPT = N_PAD // 16

_MESH = dict(core_axis_name="c", subcore_axis_name="s")


def _sc_deg_body(nchunks, epw, dst_hbm, ones_hbm, zeros_hbm, out_hbm,
                 didx_v, ones_v, acc_sh):
    cid = lax.axis_index("c")
    sid = lax.axis_index("s")
    wid = sid * 2 + cid
    pltpu.sync_copy(zeros_hbm.at[pl.ds(sid * RPT, RPT)],
                    acc_sh.at[pl.ds(sid * RPT, RPT)])
    pltpu.sync_copy(ones_hbm, ones_v)
    plsc.subcore_barrier()
    base = wid * epw

    def body(j, carry):
        off = base + j * CH
        pltpu.sync_copy(dst_hbm.at[pl.ds(off, CH)], didx_v)
        pltpu.sync_copy(ones_v, acc_sh.at[didx_v], add=True)
        return carry

    lax.fori_loop(0, nchunks, body, 0)
    plsc.subcore_barrier()
    pltpu.sync_copy(acc_sh.at[pl.ds(sid * RPT, RPT)],
                    out_hbm.at[pl.ds(cid * N_PAD + sid * RPT, RPT)])


NBUF = 2

SPLIT0 = 0.74


def _sc_gs_body(nch0, epw0, nch1, epw1, src_hbm, dst_hbm, scaled_hbm,
                zeros_hbm, out_hbm, sidx_v, didx_v, rows_v, s0, s1, acc_sh):
    cid = lax.axis_index("c")
    sid = lax.axis_index("s")
    sems = (s0, s1)
    nch = jnp.where(cid == 0, nch0, nch1)
    base = jnp.where(cid == 0, sid * epw0, 16 * epw0 + sid * epw1)
    pltpu.sync_copy(zeros_hbm.at[pl.ds(sid * RPT, RPT)],
                    acc_sh.at[pl.ds(sid * RPT, RPT)])
    plsc.subcore_barrier()

    def fire(j, b):
        pltpu.sync_copy(src_hbm.at[pl.ds(base + j * CH, CH)], sidx_v.at[b])
        pltpu.async_copy(scaled_hbm.at[sidx_v.at[b]], rows_v.at[b], sems[b])

    for b in range(NBUF - 1):
        fire(b, b)

    def outer(jo, carry):
        for b in range(NBUF):
            j = jo * NBUF + b
            jn = j + NBUF - 1
            nb = (b + NBUF - 1) % NBUF

            @pl.when(jn < nch)
            def _():
                fire(jn, nb)

            pltpu.sync_copy(dst_hbm.at[pl.ds(base + j * CH, CH)], didx_v)
            pltpu.make_async_copy(scaled_hbm.at[sidx_v.at[b]],
                                  rows_v.at[b], sems[b]).wait()
            pltpu.sync_copy(rows_v.at[b], acc_sh.at[didx_v], add=True)
        return carry

    lax.fori_loop(0, nch // NBUF, outer, 0)
    plsc.subcore_barrier()
    pltpu.sync_copy(acc_sh.at[pl.ds(sid * RPT, RPT)],
                    out_hbm.at[pl.ds(cid * N_PAD + sid * RPT, RPT)])


def _make_sc_deg(nchunks, epw):
    return pl.kernel(
        functools.partial(_sc_deg_body, nchunks, epw),
        out_type=jax.ShapeDtypeStruct((2 * N_PAD, 16), jnp.float32),
        mesh=plsc.VectorSubcoreMesh(**_MESH),
        scratch_types=[
            pltpu.VMEM((CH,), jnp.int32),
            pltpu.VMEM((CH, 16), jnp.float32),
            pltpu.VMEM_SHARED((N_PAD, 16), jnp.float32),
        ],
        compiler_params=pltpu.CompilerParams(use_tc_tiling_on_sc=False),
    )


def _make_sc_gs(nch0, epw0, nch1, epw1):
    return pl.kernel(
        functools.partial(_sc_gs_body, nch0, epw0, nch1, epw1),
        out_type=jax.ShapeDtypeStruct((2 * N_PAD, C), jnp.float32),
        mesh=plsc.VectorSubcoreMesh(**_MESH),
        scratch_types=[
            pltpu.VMEM((NBUF, CH), jnp.int32),
            pltpu.VMEM((CH,), jnp.int32),
            pltpu.VMEM((NBUF, CH, C), jnp.float32),
            pltpu.SemaphoreType.DMA,
            pltpu.SemaphoreType.DMA,
            pltpu.VMEM_SHARED((N_PAD, C), jnp.float32),
        ],
    )


def _mm_bias_body(x_ref, w_ref, b_ref, o_ref):
    o_ref[...] = jnp.dot(x_ref[...], w_ref[...],
                         preferred_element_type=jnp.float32) + b_ref[...]


def _mm_scale_body(x_ref, w_ref, s_ref, o_ref):
    o_ref[...] = jnp.dot(x_ref[...], w_ref[...],
                         preferred_element_type=jnp.float32) * s_ref[...]


def _dinv_body(d0_ref, d1_ref, o_ref):
    deg = jnp.maximum(d0_ref[...] + d1_ref[...], 1.0)
    o_ref[...] = jnp.broadcast_to(lax.rsqrt(deg)[:, :1], (B, C))


def _bn_body(has_res, r0_ref, r1_ref, s_ref, b_ref, g_ref, be_ref, res_ref,
             o_ref, sum_ref, ssq_ref):
    p = pl.program_id(0)
    i = pl.program_id(1)
    y = s_ref[...] * (r0_ref[...] + r1_ref[...]) + b_ref[...]

    @pl.when(jnp.logical_and(p == 0, i == 0))
    def _():
        sum_ref[...] = jnp.zeros_like(sum_ref)
        ssq_ref[...] = jnp.zeros_like(ssq_ref)

    @pl.when(p == 0)
    def _():
        rows = lax.broadcasted_iota(jnp.int32, (B, C), 0) + i * B
        ym = jnp.where(rows < N, y, 0.0)
        sum_ref[...] += jnp.sum(ym, axis=0, keepdims=True)
        ssq_ref[...] += jnp.sum(ym * ym, axis=0, keepdims=True)

    @pl.when(p == 1)
    def _():
        m = sum_ref[...] * (1.0 / N)
        v = ssq_ref[...] * (1.0 / N) - m * m
        h = (y - m) * lax.rsqrt(v + EPS) * g_ref[...] + be_ref[...]
        if has_res:
            h = h + res_ref[...]
        o_ref[...] = jnp.maximum(h, 0.0)


def _heads_body(h_ref, wm_ref, bm_ref, wl_ref, bl_ref, mu_ref, lv_ref):
    h = h_ref[...]
    mu_ref[...] = jnp.dot(h, wm_ref[...],
                          preferred_element_type=jnp.float32) + bm_ref[...]
    lv_ref[...] = jnp.dot(h, wl_ref[...],
                          preferred_element_type=jnp.float32) + bl_ref[...]


_row_spec = pl.BlockSpec((B, C), lambda i: (i, 0))
_full_w = pl.BlockSpec((C, C), lambda i: (0, 0))
_full_b = pl.BlockSpec((1, C), lambda i: (0, 0))
_out_f32 = jax.ShapeDtypeStruct((N_PAD, C), jnp.float32)


def _mm_bias(x, w, b):
    return pl.pallas_call(
        _mm_bias_body, grid=(NB,),
        in_specs=[_row_spec, _full_w, _full_b],
        out_specs=_row_spec, out_shape=_out_f32,
    )(x, w, b.reshape(1, C))


def _mm_scale(x, w, s):
    return pl.pallas_call(
        _mm_scale_body, grid=(NB,),
        in_specs=[_row_spec, _full_w, _row_spec],
        out_specs=_row_spec, out_shape=_out_f32,
    )(x, w, s)


def _dinv(deg2):
    spec16 = pl.BlockSpec((B, 16), lambda i: (i, 0))
    return pl.pallas_call(
        _dinv_body, grid=(NB,),
        in_specs=[spec16, spec16],
        out_specs=_row_spec, out_shape=_out_f32,
    )(deg2[:N_PAD], deg2[N_PAD:])


def _bn(raw2, dinv, b, g, be, res):
    has_res = res is not None
    spec2 = pl.BlockSpec((B, C), lambda p, i: (i, 0))
    full2 = pl.BlockSpec((1, C), lambda p, i: (0, 0))
    out2 = pl.BlockSpec((B, C), lambda p, i: (jnp.where(p == 1, i, 0), 0))
    if not has_res:
        res = raw2[:N_PAD]
    return pl.pallas_call(
        functools.partial(_bn_body, has_res), grid=(2, NB),
        in_specs=[spec2, spec2, spec2, full2, full2, full2, spec2],
        out_specs=out2, out_shape=_out_f32,
        scratch_shapes=[pltpu.VMEM((1, C), jnp.float32),
                        pltpu.VMEM((1, C), jnp.float32)],
    )(raw2[:N_PAD], raw2[N_PAD:], dinv, b.reshape(1, C), g.reshape(1, C),
      be.reshape(1, C), res)


def _heads(h, wm, bm, wl, bl):
    return pl.pallas_call(
        _heads_body, grid=(NB,),
        in_specs=[_row_spec, _full_w, _full_b, _full_w, _full_b],
        out_specs=(_row_spec, _row_spec), out_shape=(_out_f32, _out_f32),
    )(h, wm, bm.reshape(1, C), wl, bl.reshape(1, C))


def kernel(x, edge_index, W1, b1, W2, b2, W3, b3, W4, b4,
           g1, be1, g2, be2, g3, be3, g4, be4,
           Wr, br, Wmu, bmu, Wlv, blv):
    e = edge_index.shape[1]
    e2 = e + N
    grain = CH * NBUF
    epw0 = max(grain, int(round(e2 * SPLIT0 / 16 / grain)) * grain)
    epw1 = max(grain, -(-(e2 - 16 * epw0) // (16 * grain)) * grain)
    nch0, nch1 = epw0 // CH, epw1 // CH
    e_pad = 16 * (epw0 + epw1)
    pad = e_pad - e2
    epw_deg = e_pad // NW
    nch_deg = epw_deg // CH

    loop = jnp.arange(N, dtype=jnp.int32)
    src = jnp.concatenate([edge_index[0].astype(jnp.int32), loop,
                           jnp.zeros((pad,), jnp.int32)])
    dst = jnp.concatenate([edge_index[1].astype(jnp.int32), loop,
                           jnp.full((pad,), JUNK, jnp.int32)])

    zeros_big = jnp.zeros((N_PAD, C), jnp.float32)
    x_pad = jnp.zeros((N_PAD, C), x.dtype).at[:N].set(x)

    deg2 = _make_sc_deg(nch_deg, epw_deg)(
        dst, jnp.ones((CH, 16), jnp.float32), jnp.zeros((N_PAD, 16),
                                                        jnp.float32))
    dinv = _dinv(deg2)

    identity = _mm_bias(x_pad, Wr, br)

    sc_gs = _make_sc_gs(nch0, epw0, nch1, epw1)
    h = x_pad
    for W, b, g, be, res in ((W1, b1, g1, be1, None),
                             (W2, b2, g2, be2, None),
                             (W3, b3, g3, be3, None),
                             (W4, b4, g4, be4, identity)):
        scaled = _mm_scale(h, W, dinv)
        raw2 = sc_gs(src, dst, scaled, zeros_big)
        h = _bn(raw2, dinv, b, g, be, res)

    mu, lv = _heads(h, Wmu, bmu, Wlv, blv)
    return mu[:N], lv[:N]


# speedup vs baseline: 2.0749x; 1.1171x over previous
import functools

import jax
import jax.numpy as jnp
from jax import lax
from jax.experimental import pallas as pl
from jax.experimental.pallas import tpu as pltpu
from jax.experimental.pallas import tpu_sc as plsc

N = 10000
C = 128
EPS = 1e-5
N_PAD = 10240
JUNK = N
B = 1024
NB = N_PAD // B

NW = 32
CH = 128
RPT = N_PAD // 16

_MESH = dict(core_axis_name="c", subcore_axis_name="s")


def _sc_deg_body(nchunks, epw, dst_hbm, ones_hbm, zeros_hbm, out_hbm,
                 didx_v, ones_v, s0, s1, acc_sh):
    cid = lax.axis_index("c")
    sid = lax.axis_index("s")
    wid = sid * 2 + cid
    sems = (s0, s1)
    base = wid * epw
    pltpu.sync_copy(zeros_hbm.at[pl.ds(sid * RPT, RPT)],
                    acc_sh.at[pl.ds(sid * RPT, RPT)])
    pltpu.sync_copy(ones_hbm, ones_v)
    plsc.subcore_barrier()

    def fire(j, b):
        pltpu.async_copy(dst_hbm.at[pl.ds(base + j * CH, CH)],
                         didx_v.at[b], sems[b])

    fire(0, 0)

    def body(jo, carry):
        for b in range(2):
            j = jo * 2 + b

            @pl.when(j + 1 < nchunks)
            def _():
                fire(j + 1, (b + 1) % 2)

            pltpu.make_async_copy(dst_hbm.at[pl.ds(base + j * CH, CH)],
                                  didx_v.at[b], sems[b]).wait()
            pltpu.sync_copy(ones_v, acc_sh.at[didx_v.at[b]], add=True)
        return carry

    lax.fori_loop(0, nchunks // 2, body, 0)
    if nchunks % 2:
        j = nchunks - 1
        pltpu.make_async_copy(dst_hbm.at[pl.ds(base + j * CH, CH)],
                              didx_v.at[0], sems[0]).wait()
        pltpu.sync_copy(ones_v, acc_sh.at[didx_v.at[0]], add=True)
    plsc.subcore_barrier()
    pltpu.sync_copy(acc_sh.at[pl.ds(sid * RPT, RPT)],
                    out_hbm.at[pl.ds(cid * N_PAD + sid * RPT, RPT)])


NBUF = 2

SPLIT0 = 0.70


def _sc_gs_body(nch0, epw0, nch1, epw1, src_hbm, dst_hbm, scaled_hbm,
                zeros_hbm, out_hbm, sidx_v, didx_v, rows_v, s0, s1, acc_sh):
    cid = lax.axis_index("c")
    sid = lax.axis_index("s")
    sems = (s0, s1)
    nch = jnp.where(cid == 0, nch0, nch1)
    base = jnp.where(cid == 0, sid * epw0, 16 * epw0 + sid * epw1)
    pltpu.sync_copy(zeros_hbm.at[pl.ds(sid * RPT, RPT)],
                    acc_sh.at[pl.ds(sid * RPT, RPT)])
    plsc.subcore_barrier()

    def fire(j, b):
        pltpu.sync_copy(src_hbm.at[pl.ds(base + j * CH, CH)], sidx_v.at[b])
        pltpu.async_copy(scaled_hbm.at[sidx_v.at[b]], rows_v.at[b], sems[b])

    for b in range(NBUF - 1):
        fire(b, b)

    def outer(jo, carry):
        for b in range(NBUF):
            j = jo * NBUF + b
            jn = j + NBUF - 1
            nb = (b + NBUF - 1) % NBUF

            @pl.when(jn < nch)
            def _():
                fire(jn, nb)

            pltpu.sync_copy(dst_hbm.at[pl.ds(base + j * CH, CH)], didx_v)
            pltpu.make_async_copy(scaled_hbm.at[sidx_v.at[b]],
                                  rows_v.at[b], sems[b]).wait()
            pltpu.sync_copy(rows_v.at[b], acc_sh.at[didx_v], add=True)
        return carry

    lax.fori_loop(0, nch // NBUF, outer, 0)
    plsc.subcore_barrier()
    pltpu.sync_copy(acc_sh.at[pl.ds(sid * RPT, RPT)],
                    out_hbm.at[pl.ds(cid * N_PAD + sid * RPT, RPT)])


def _make_sc_deg(nchunks, epw):
    return pl.kernel(
        functools.partial(_sc_deg_body, nchunks, epw),
        out_type=jax.ShapeDtypeStruct((2 * N_PAD, 16), jnp.float32),
        mesh=plsc.VectorSubcoreMesh(**_MESH),
        scratch_types=[
            pltpu.VMEM((2, CH), jnp.int32),
            pltpu.VMEM((CH, 16), jnp.float32),
            pltpu.SemaphoreType.DMA,
            pltpu.SemaphoreType.DMA,
            pltpu.VMEM_SHARED((N_PAD, 16), jnp.float32),
        ],
        compiler_params=pltpu.CompilerParams(use_tc_tiling_on_sc=False),
    )


def _make_sc_gs(nch0, epw0, nch1, epw1):
    return pl.kernel(
        functools.partial(_sc_gs_body, nch0, epw0, nch1, epw1),
        out_type=jax.ShapeDtypeStruct((2 * N_PAD, C), jnp.float32),
        mesh=plsc.VectorSubcoreMesh(**_MESH),
        scratch_types=[
            pltpu.VMEM((NBUF, CH), jnp.int32),
            pltpu.VMEM((CH,), jnp.int32),
            pltpu.VMEM((NBUF, CH, C), jnp.float32),
            pltpu.SemaphoreType.DMA,
            pltpu.SemaphoreType.DMA,
            pltpu.VMEM_SHARED((N_PAD, C), jnp.float32),
        ],
    )


def _mm_bias_body(x_ref, w_ref, b_ref, o_ref):
    o_ref[...] = jnp.dot(x_ref[...], w_ref[...],
                         preferred_element_type=jnp.float32) + b_ref[...]


def _mm_scale_body(x_ref, w_ref, s_ref, o_ref):
    o_ref[...] = jnp.dot(x_ref[...], w_ref[...],
                         preferred_element_type=jnp.float32) * s_ref[...]


def _dinv_body(d0_ref, d1_ref, o_ref):
    deg = jnp.maximum(d0_ref[...] + d1_ref[...], 1.0)
    o_ref[...] = jnp.broadcast_to(lax.rsqrt(deg)[:, :1], (B, C))


def _bn_body(has_res, r0_ref, r1_ref, s_ref, b_ref, g_ref, be_ref, res_ref,
             o_ref, sum_ref, ssq_ref):
    p = pl.program_id(0)
    i = pl.program_id(1)
    y = s_ref[...] * (r0_ref[...] + r1_ref[...]) + b_ref[...]

    @pl.when(jnp.logical_and(p == 0, i == 0))
    def _():
        sum_ref[...] = jnp.zeros_like(sum_ref)
        ssq_ref[...] = jnp.zeros_like(ssq_ref)

    @pl.when(p == 0)
    def _():
        rows = lax.broadcasted_iota(jnp.int32, (B, C), 0) + i * B
        ym = jnp.where(rows < N, y, 0.0)
        sum_ref[...] += jnp.sum(ym, axis=0, keepdims=True)
        ssq_ref[...] += jnp.sum(ym * ym, axis=0, keepdims=True)

    @pl.when(p == 1)
    def _():
        m = sum_ref[...] * (1.0 / N)
        v = ssq_ref[...] * (1.0 / N) - m * m
        h = (y - m) * lax.rsqrt(v + EPS) * g_ref[...] + be_ref[...]
        if has_res:
            h = h + res_ref[...]
        o_ref[...] = jnp.maximum(h, 0.0)


def _bn_mm_body(r0_ref, r1_ref, s_ref, b_ref, g_ref, be_ref, w_ref,
                o_ref, sum_ref, ssq_ref):
    p = pl.program_id(0)
    i = pl.program_id(1)
    y = s_ref[...] * (r0_ref[...] + r1_ref[...]) + b_ref[...]

    @pl.when(jnp.logical_and(p == 0, i == 0))
    def _():
        sum_ref[...] = jnp.zeros_like(sum_ref)
        ssq_ref[...] = jnp.zeros_like(ssq_ref)

    @pl.when(p == 0)
    def _():
        rows = lax.broadcasted_iota(jnp.int32, (B, C), 0) + i * B
        ym = jnp.where(rows < N, y, 0.0)
        sum_ref[...] += jnp.sum(ym, axis=0, keepdims=True)
        ssq_ref[...] += jnp.sum(ym * ym, axis=0, keepdims=True)

    @pl.when(p == 1)
    def _():
        m = sum_ref[...] * (1.0 / N)
        v = ssq_ref[...] * (1.0 / N) - m * m
        h = jnp.maximum((y - m) * lax.rsqrt(v + EPS) * g_ref[...]
                        + be_ref[...], 0.0)
        o_ref[...] = jnp.dot(h, w_ref[...],
                             preferred_element_type=jnp.float32) * s_ref[...]


def _bn_heads_body(r0_ref, r1_ref, s_ref, b_ref, g_ref, be_ref, res_ref,
                   wm_ref, bm_ref, wl_ref, bl_ref, mu_ref, lv_ref,
                   sum_ref, ssq_ref):
    p = pl.program_id(0)
    i = pl.program_id(1)
    y = s_ref[...] * (r0_ref[...] + r1_ref[...]) + b_ref[...]

    @pl.when(jnp.logical_and(p == 0, i == 0))
    def _():
        sum_ref[...] = jnp.zeros_like(sum_ref)
        ssq_ref[...] = jnp.zeros_like(ssq_ref)

    @pl.when(p == 0)
    def _():
        rows = lax.broadcasted_iota(jnp.int32, (B, C), 0) + i * B
        ym = jnp.where(rows < N, y, 0.0)
        sum_ref[...] += jnp.sum(ym, axis=0, keepdims=True)
        ssq_ref[...] += jnp.sum(ym * ym, axis=0, keepdims=True)

    @pl.when(p == 1)
    def _():
        m = sum_ref[...] * (1.0 / N)
        v = ssq_ref[...] * (1.0 / N) - m * m
        h = jnp.maximum((y - m) * lax.rsqrt(v + EPS) * g_ref[...]
                        + be_ref[...] + res_ref[...], 0.0)
        mu_ref[...] = jnp.dot(h, wm_ref[...],
                              preferred_element_type=jnp.float32) + bm_ref[...]
        lv_ref[...] = jnp.dot(h, wl_ref[...],
                              preferred_element_type=jnp.float32) + bl_ref[...]


def _heads_body(h_ref, wm_ref, bm_ref, wl_ref, bl_ref, mu_ref, lv_ref):
    h = h_ref[...]
    mu_ref[...] = jnp.dot(h, wm_ref[...],
                          preferred_element_type=jnp.float32) + bm_ref[...]
    lv_ref[...] = jnp.dot(h, wl_ref[...],
                          preferred_element_type=jnp.float32) + bl_ref[...]


_row_spec = pl.BlockSpec((B, C), lambda i: (i, 0))
_full_w = pl.BlockSpec((C, C), lambda i: (0, 0))
_full_b = pl.BlockSpec((1, C), lambda i: (0, 0))
_out_f32 = jax.ShapeDtypeStruct((N_PAD, C), jnp.float32)


def _mm_bias(x, w, b):
    return pl.pallas_call(
        _mm_bias_body, grid=(NB,),
        in_specs=[_row_spec, _full_w, _full_b],
        out_specs=_row_spec, out_shape=_out_f32,
    )(x, w, b.reshape(1, C))


def _mm_scale(x, w, s):
    return pl.pallas_call(
        _mm_scale_body, grid=(NB,),
        in_specs=[_row_spec, _full_w, _row_spec],
        out_specs=_row_spec, out_shape=_out_f32,
    )(x, w, s)


def _dinv(deg2):
    spec16 = pl.BlockSpec((B, 16), lambda i: (i, 0))
    return pl.pallas_call(
        _dinv_body, grid=(NB,),
        in_specs=[spec16, spec16],
        out_specs=_row_spec, out_shape=_out_f32,
    )(deg2[:N_PAD], deg2[N_PAD:])


def _bn(raw2, dinv, b, g, be, res):
    has_res = res is not None
    spec2 = pl.BlockSpec((B, C), lambda p, i: (i, 0))
    full2 = pl.BlockSpec((1, C), lambda p, i: (0, 0))
    out2 = pl.BlockSpec((B, C), lambda p, i: (jnp.where(p == 1, i, 0), 0))
    if not has_res:
        res = raw2[:N_PAD]
    return pl.pallas_call(
        functools.partial(_bn_body, has_res), grid=(2, NB),
        in_specs=[spec2, spec2, spec2, full2, full2, full2, spec2],
        out_specs=out2, out_shape=_out_f32,
        scratch_shapes=[pltpu.VMEM((1, C), jnp.float32),
                        pltpu.VMEM((1, C), jnp.float32)],
    )(raw2[:N_PAD], raw2[N_PAD:], dinv, b.reshape(1, C), g.reshape(1, C),
      be.reshape(1, C), res)


def _bn_mm(raw2, dinv, b, g, be, w):
    spec2 = pl.BlockSpec((B, C), lambda p, i: (i, 0))
    full2 = pl.BlockSpec((1, C), lambda p, i: (0, 0))
    fullw = pl.BlockSpec((C, C), lambda p, i: (0, 0))
    out2 = pl.BlockSpec((B, C), lambda p, i: (jnp.where(p == 1, i, 0), 0))
    return pl.pallas_call(
        _bn_mm_body, grid=(2, NB),
        in_specs=[spec2, spec2, spec2, full2, full2, full2, fullw],
        out_specs=out2, out_shape=_out_f32,
        scratch_shapes=[pltpu.VMEM((1, C), jnp.float32),
                        pltpu.VMEM((1, C), jnp.float32)],
    )(raw2[:N_PAD], raw2[N_PAD:], dinv, b.reshape(1, C), g.reshape(1, C),
      be.reshape(1, C), w)


def _bn_heads(raw2, dinv, b, g, be, res, wm, bm, wl, bl):
    spec2 = pl.BlockSpec((B, C), lambda p, i: (i, 0))
    full2 = pl.BlockSpec((1, C), lambda p, i: (0, 0))
    fullw = pl.BlockSpec((C, C), lambda p, i: (0, 0))
    out2 = pl.BlockSpec((B, C), lambda p, i: (jnp.where(p == 1, i, 0), 0))
    return pl.pallas_call(
        _bn_heads_body, grid=(2, NB),
        in_specs=[spec2, spec2, spec2, full2, full2, full2, spec2,
                  fullw, full2, fullw, full2],
        out_specs=(out2, out2), out_shape=(_out_f32, _out_f32),
        scratch_shapes=[pltpu.VMEM((1, C), jnp.float32),
                        pltpu.VMEM((1, C), jnp.float32)],
    )(raw2[:N_PAD], raw2[N_PAD:], dinv, b.reshape(1, C), g.reshape(1, C),
      be.reshape(1, C), res, wm, bm.reshape(1, C), wl, bl.reshape(1, C))


def _heads(h, wm, bm, wl, bl):
    return pl.pallas_call(
        _heads_body, grid=(NB,),
        in_specs=[_row_spec, _full_w, _full_b, _full_w, _full_b],
        out_specs=(_row_spec, _row_spec), out_shape=(_out_f32, _out_f32),
    )(h, wm, bm.reshape(1, C), wl, bl.reshape(1, C))


def kernel(x, edge_index, W1, b1, W2, b2, W3, b3, W4, b4,
           g1, be1, g2, be2, g3, be3, g4, be4,
           Wr, br, Wmu, bmu, Wlv, blv):
    e = edge_index.shape[1]
    e2 = e + N
    grain = CH * NBUF
    epw0 = max(grain, int(round(e2 * SPLIT0 / 16 / grain)) * grain)
    epw1 = max(grain, -(-(e2 - 16 * epw0) // (16 * grain)) * grain)
    nch0, nch1 = epw0 // CH, epw1 // CH
    e_pad = 16 * (epw0 + epw1)
    pad = e_pad - e2
    epw_deg = e_pad // NW
    nch_deg = epw_deg // CH

    loop = jnp.arange(N, dtype=jnp.int32)
    src = jnp.concatenate([edge_index[0].astype(jnp.int32), loop,
                           jnp.zeros((pad,), jnp.int32)])
    dst = jnp.concatenate([edge_index[1].astype(jnp.int32), loop,
                           jnp.full((pad,), JUNK, jnp.int32)])

    zeros_big = jnp.zeros((N_PAD, C), jnp.float32)
    x_pad = jnp.zeros((N_PAD, C), x.dtype).at[:N].set(x)

    deg2 = _make_sc_deg(nch_deg, epw_deg)(
        dst, jnp.ones((CH, 16), jnp.float32), jnp.zeros((N_PAD, 16),
                                                        jnp.float32))
    dinv = _dinv(deg2)

    identity = _mm_bias(x_pad, Wr, br)

    sc_gs = _make_sc_gs(nch0, epw0, nch1, epw1)
    scaled = _mm_scale(x_pad, W1, dinv)
    for b, g, be, Wn in ((b1, g1, be1, W2), (b2, g2, be2, W3),
                         (b3, g3, be3, W4)):
        raw2 = sc_gs(src, dst, scaled, zeros_big)
        scaled = _bn_mm(raw2, dinv, b, g, be, Wn)
    raw2 = sc_gs(src, dst, scaled, zeros_big)
    mu, lv = _bn_heads(raw2, dinv, b4, g4, be4, identity,
                       Wmu, bmu, Wlv, blv)
    return mu[:N], lv[:N]
